# Initial kernel scaffold; baseline (speedup 1.0000x reference)
import jax
import jax.numpy as jnp
from jax.experimental import pallas as pl


def kernel(face_preds, landmark_preds, gaze_preds, boxes, landmarks, gaze, matches, labels):
    raise NotImplementedError("write your pallas kernel here")


# speedup vs baseline: 21.0166x; 21.0166x over previous
import functools

import jax
import jax.numpy as jnp
from jax import lax
from jax.experimental import pallas as pl
from jax.experimental.pallas import tpu as pltpu
from jax.experimental.pallas import tpu_sc as plsc

B = 16
N = 16384
M = 64
L = 16
NC = 2
NS = 16
NW = NC * NS
APW = (B * N) // NW
CH = 2048
NCHUNK = APW // CH
GRP = CH // L

_mesh = plsc.VectorSubcoreMesh(core_axis_name="c", subcore_axis_name="s")


def _body(face_h, lmp_h, gzp_h, tbox_h, tlm_h, tgz_h, mat_h, lab_h, out_h,
          face_v0, face_v1, lmp_v0, lmp_v1, gzp_v0, gzp_v1,
          mat_v0, mat_v1, lab_v0, lab_v1,
          tbox_v, tlm_v, tgz_v, out_v, sem0, sem1):
    cid = lax.axis_index("c")
    sid = lax.axis_index("s")
    wid = sid * NC + cid
    img = wid // 2
    half = wid % 2
    a0 = img * N + half * APW

    pltpu.sync_copy(tbox_h.at[pl.ds(img * (M * 4), M * 4)], tbox_v)
    pltpu.sync_copy(tlm_h.at[pl.ds(img * (M * 10), M * 10)], tlm_v)
    pltpu.sync_copy(tgz_h.at[pl.ds(img * (M * 2), M * 2)], tgz_v)

    bufs = ((face_v0, lmp_v0, gzp_v0, mat_v0, lab_v0, sem0),
            (face_v1, lmp_v1, gzp_v1, mat_v1, lab_v1, sem1))

    def start(c, slot):
        fv, lv, gv, mv, bv, sem = bufs[slot]
        base = a0 + c * CH
        return [
            pltpu.async_copy(face_h.at[pl.ds(base * 5, CH * 5)], fv, sem),
            pltpu.async_copy(lmp_h.at[pl.ds(base * 10, CH * 10)], lv, sem),
            pltpu.async_copy(gzp_h.at[pl.ds(base * 2, CH * 2)], gv, sem),
            pltpu.async_copy(mat_h.at[pl.ds(base, CH)], mv, sem),
            pltpu.async_copy(lab_h.at[pl.ds(base, CH)], bv, sem),
        ]

    iota = jnp.arange(L, dtype=jnp.int32)
    i5 = iota * 5
    i10 = iota * 10
    i2 = iota * 2

    def compute(slot, accs):
        fv, lv, gv, mv, bv, _ = bufs[slot]

        def group(g, accs):
            abce, abox, alm, agz = accs
            off = (g * L).astype(jnp.int32)
            aidx = off + iota
            m = plsc.load_gather(mv, [aidx])
            lab = plsc.load_gather(bv, [aidx])
            maskf = jnp.where(lab > 0.0, 1.0, 0.0).astype(jnp.float32)

            def sl1(pred_ref, p_base, p_stride_iota, tbl_ref, t_base, ncomp):
                sq = jnp.zeros((L,), jnp.float32)
                mx = jnp.zeros((L,), jnp.float32)
                for j in range(ncomp):
                    p = plsc.load_gather(pred_ref, [p_base + p_stride_iota + j])
                    t = plsc.load_gather(tbl_ref, [t_base + j])
                    d = jnp.abs(p - t)
                    dm = jnp.minimum(d, 1.0)
                    sq = sq + dm * dm
                    mx = mx + jnp.maximum(d, 1.0)
                return (0.5 * sq + mx - float(ncomp)) * maskf

            abox = abox + sl1(fv, off * 5, i5, tbox_v, m * 4, 4)
            alm = alm + sl1(lv, off * 10, i10, tlm_v, m * 10, 10)
            agz = agz + sl1(gv, off * 2, i2, tgz_v, m * 2, 2)

            x = plsc.load_gather(fv, [off * 5 + i5 + 4])
            u = jnp.exp(-jnp.abs(x))
            v = u / (u + 2.0)
            v2 = v * v
            sp = v * (2.0 + v2 * (2.0 / 3.0 + v2 * (2.0 / 5.0
                      + v2 * (2.0 / 7.0 + v2 * (2.0 / 9.0)))))
            abce = abce + (jnp.maximum(x, 0.0) - x * lab + sp)
            return (abce, abox, alm, agz)

        return lax.fori_loop(0, GRP, group, accs)

    z = jnp.zeros((L,), jnp.float32)
    accs = (z, z, z, z)
    pending = start(0, 0)
    for c in range(NCHUNK):
        for hd in pending:
            hd.wait()
        if c + 1 < NCHUNK:
            nxt = start(c + 1, (c + 1) % 2)
        else:
            nxt = []
        accs = compute(c % 2, accs)
        pending = nxt

    out_v[0, :] = accs[0]
    out_v[1, :] = accs[1]
    out_v[2, :] = accs[2]
    out_v[3, :] = accs[3]
    pltpu.sync_copy(out_v, out_h.at[wid])


_sc_loss = functools.partial(
    pl.kernel,
    out_type=jax.ShapeDtypeStruct((NW, 4, L), jnp.float32),
    mesh=_mesh,
    scratch_types=[
        pltpu.VMEM((CH * 5,), jnp.float32),
        pltpu.VMEM((CH * 5,), jnp.float32),
        pltpu.VMEM((CH * 10,), jnp.float32),
        pltpu.VMEM((CH * 10,), jnp.float32),
        pltpu.VMEM((CH * 2,), jnp.float32),
        pltpu.VMEM((CH * 2,), jnp.float32),
        pltpu.VMEM((CH,), jnp.int32),
        pltpu.VMEM((CH,), jnp.int32),
        pltpu.VMEM((CH,), jnp.float32),
        pltpu.VMEM((CH,), jnp.float32),
        pltpu.VMEM((M * 4,), jnp.float32),
        pltpu.VMEM((M * 10,), jnp.float32),
        pltpu.VMEM((M * 2,), jnp.float32),
        pltpu.VMEM((4, L), jnp.float32),
        pltpu.SemaphoreType.DMA,
        pltpu.SemaphoreType.DMA,
    ],
    compiler_params=pltpu.CompilerParams(needs_layout_passes=False),
)(_body)


def kernel(face_preds, landmark_preds, gaze_preds, boxes, landmarks, gaze,
           matches, labels):
    part = _sc_loss(
        face_preds.reshape(-1),
        landmark_preds.reshape(-1),
        gaze_preds.reshape(-1),
        boxes.reshape(-1),
        landmarks.reshape(-1),
        gaze.reshape(-1),
        matches.reshape(-1).astype(jnp.int32),
        labels.reshape(-1),
    )
    s = jnp.sum(part, axis=(0, 2))
    face_loss = s[0] + s[1]
    landmark_loss = s[2]
    gaze_loss = s[3]
    total_loss = face_loss + landmark_loss + gaze_loss
    return (total_loss, face_loss, landmark_loss, gaze_loss)
